# baseline (speedup 1.0000x reference)
import functools

import jax
import jax.numpy as jnp
from jax import lax
from jax.experimental import pallas as pl
from jax.experimental.pallas import tpu as pltpu
from jax.experimental.pallas import tpu_sc as plsc

_CHUNK_ROWS = 32
_NBUF = 4


def kernel(input_ids, weights):
    seq_len = input_ids.shape[-1]
    _, dim = weights.shape

    info = plsc.get_sparse_core_info()
    num_cores, num_subcores = info.num_cores, info.num_subcores
    num_workers = num_cores * num_subcores
    rows_per_worker = seq_len // num_workers
    assert rows_per_worker * num_workers == seq_len
    n_chunks = rows_per_worker // _CHUNK_ROWS
    assert n_chunks * _CHUNK_ROWS == rows_per_worker and n_chunks >= _NBUF

    mesh = plsc.VectorSubcoreMesh(core_axis_name="c", subcore_axis_name="s")

    @functools.partial(
        pl.kernel,
        mesh=mesh,
        out_type=jax.ShapeDtypeStruct((seq_len, dim), weights.dtype),
        scratch_types=(
            [pltpu.VMEM((_CHUNK_ROWS, dim), jnp.float32)] * _NBUF
            + [pltpu.SemaphoreType.DMA] * (2 * _NBUF)
        ),
    )
    def gather_rows(w_hbm, out_hbm, *scratch):
        bufs = scratch[:_NBUF]
        sins = scratch[_NBUF : 2 * _NBUF]
        souts = scratch[2 * _NBUF :]
        wid = lax.axis_index("s") * num_cores + lax.axis_index("c")
        base = wid * rows_per_worker

        def start_in(k):
            return pltpu.async_copy(
                w_hbm.at[pl.ds(base + k * _CHUNK_ROWS, _CHUNK_ROWS)],
                bufs[k % _NBUF],
                sins[k % _NBUF],
            )

        def start_out(k):
            return pltpu.async_copy(
                bufs[k % _NBUF],
                out_hbm.at[pl.ds(base + k * _CHUNK_ROWS, _CHUNK_ROWS)],
                souts[k % _NBUF],
            )

        lookahead = _NBUF - 2
        in_cp = [None] * n_chunks
        out_cp = [None] * n_chunks
        for k in range(lookahead):
            in_cp[k] = start_in(k)
        for k in range(n_chunks):
            if k + lookahead < n_chunks:
                if k + lookahead - _NBUF >= 0:
                    out_cp[k + lookahead - _NBUF].wait()
                in_cp[k + lookahead] = start_in(k + lookahead)
            in_cp[k].wait()
            out_cp[k] = start_out(k)
        for k in range(max(0, n_chunks - _NBUF), n_chunks):
            if out_cp[k] is not None:
                out_cp[k].wait()

    return gather_rows(weights)


# speedup vs baseline: 1.0026x; 1.0026x over previous
import functools

import jax
import jax.numpy as jnp
from jax import lax
from jax.experimental import pallas as pl
from jax.experimental.pallas import tpu as pltpu
from jax.experimental.pallas import tpu_sc as plsc

_CHUNK_ROWS = 16
_NBUF = 8


def kernel(input_ids, weights):
    seq_len = input_ids.shape[-1]
    _, dim = weights.shape

    info = plsc.get_sparse_core_info()
    num_cores, num_subcores = info.num_cores, info.num_subcores
    num_workers = num_cores * num_subcores
    rows_per_worker = seq_len // num_workers
    assert rows_per_worker * num_workers == seq_len
    n_chunks = rows_per_worker // _CHUNK_ROWS
    assert n_chunks * _CHUNK_ROWS == rows_per_worker and n_chunks >= _NBUF

    mesh = plsc.VectorSubcoreMesh(core_axis_name="c", subcore_axis_name="s")

    @functools.partial(
        pl.kernel,
        mesh=mesh,
        out_type=jax.ShapeDtypeStruct((seq_len, dim), weights.dtype),
        scratch_types=(
            [pltpu.VMEM((_CHUNK_ROWS, dim), jnp.float32)] * _NBUF
            + [pltpu.SemaphoreType.DMA] * (2 * _NBUF)
        ),
    )
    def gather_rows(w_hbm, out_hbm, *scratch):
        bufs = scratch[:_NBUF]
        sins = scratch[_NBUF : 2 * _NBUF]
        souts = scratch[2 * _NBUF :]
        wid = lax.axis_index("s") * num_cores + lax.axis_index("c")
        base = wid * rows_per_worker

        def start_in(k):
            return pltpu.async_copy(
                w_hbm.at[pl.ds(base + k * _CHUNK_ROWS, _CHUNK_ROWS)],
                bufs[k % _NBUF],
                sins[k % _NBUF],
            )

        def start_out(k):
            return pltpu.async_copy(
                bufs[k % _NBUF],
                out_hbm.at[pl.ds(base + k * _CHUNK_ROWS, _CHUNK_ROWS)],
                souts[k % _NBUF],
            )

        lookahead = _NBUF - 2
        in_cp = [None] * n_chunks
        out_cp = [None] * n_chunks
        for k in range(lookahead):
            in_cp[k] = start_in(k)
        for k in range(n_chunks):
            if k + lookahead < n_chunks:
                if k + lookahead - _NBUF >= 0:
                    out_cp[k + lookahead - _NBUF].wait()
                in_cp[k + lookahead] = start_in(k + lookahead)
            in_cp[k].wait()
            out_cp[k] = start_out(k)
        for k in range(max(0, n_chunks - _NBUF), n_chunks):
            if out_cp[k] is not None:
                out_cp[k].wait()

    return gather_rows(weights)


# speedup vs baseline: 1.0134x; 1.0107x over previous
import functools

import jax
import jax.numpy as jnp
from jax import lax
from jax.experimental import pallas as pl
from jax.experimental.pallas import tpu as pltpu
from jax.experimental.pallas import tpu_sc as plsc

_CHUNK_ROWS = 32
_NBUF = 4


def kernel(input_ids, weights):
    seq_len = input_ids.shape[-1]
    _, dim = weights.shape

    info = plsc.get_sparse_core_info()
    num_cores, num_subcores = info.num_cores, info.num_subcores
    num_workers = num_cores * num_subcores
    rows_per_worker = seq_len // num_workers
    assert rows_per_worker * num_workers == seq_len
    n_chunks = rows_per_worker // _CHUNK_ROWS
    assert n_chunks * _CHUNK_ROWS == rows_per_worker and n_chunks >= _NBUF

    mesh = plsc.VectorSubcoreMesh(core_axis_name="c", subcore_axis_name="s")

    @functools.partial(
        pl.kernel,
        mesh=mesh,
        out_type=jax.ShapeDtypeStruct((seq_len, dim), weights.dtype),
        scratch_types=(
            [pltpu.VMEM((_CHUNK_ROWS, dim), jnp.float32)] * _NBUF
            + [pltpu.SemaphoreType.DMA] * (2 * _NBUF)
        ),
    )
    def gather_rows(w_hbm, out_hbm, *scratch):
        bufs = scratch[:_NBUF]
        sins = scratch[_NBUF : 2 * _NBUF]
        souts = scratch[2 * _NBUF :]
        wid = lax.axis_index("s") * num_cores + lax.axis_index("c")
        base = wid * rows_per_worker

        def start_in(k):
            return pltpu.async_copy(
                w_hbm.at[pl.ds(base + k * _CHUNK_ROWS, _CHUNK_ROWS)],
                bufs[k % _NBUF],
                sins[k % _NBUF],
            )

        def start_out(k):
            return pltpu.async_copy(
                bufs[k % _NBUF],
                out_hbm.at[pl.ds(base + k * _CHUNK_ROWS, _CHUNK_ROWS)],
                souts[k % _NBUF],
            )

        in_cp = [None] * n_chunks
        out_cp = [None] * n_chunks
        for k in range(_NBUF - 1):
            in_cp[k] = start_in(k)
        for k in range(n_chunks):
            in_cp[k].wait()
            nxt = k + _NBUF - 1
            if nxt < n_chunks:
                if nxt - _NBUF >= 0:
                    out_cp[nxt - _NBUF].wait()
                in_cp[nxt] = start_in(nxt)
            out_cp[k] = start_out(k)
        for k in range(max(0, n_chunks - _NBUF), n_chunks):
            if out_cp[k] is not None:
                out_cp[k].wait()

    return gather_rows(weights)
